# baseline (speedup 1.0000x reference)
import functools

import jax
import jax.numpy as jnp
from jax import lax
from jax.experimental import pallas as pl
from jax.experimental.pallas import tpu as pltpu
from jax.experimental.pallas import tpu_sc as plsc

B, N, T, P, D, O = 16, 2048, 1, 6, 128, 128
BN = B * N
BLK = 512
NPB = N // BLK

NC, NS = 2, 16
NW = NC * NS
RPW = BN // NW
G = 16
NCH = RPW // G
NPAIR = NCH // 2


def _sc_body(embs, aux_hbm, out_hbm,
             ebuf0, ebuf1, obuf0, obuf1, auxv, esem, osem):
    ci = lax.axis_index("c")
    si = lax.axis_index("s")
    wid = si * NC + ci
    rowbase = wid * RPW
    bi = rowbase // N
    n0 = rowbase % N

    ebufs = (ebuf0, ebuf1)
    obufs = (obuf0, obuf1)

    pltpu.sync_copy(aux_hbm.at[:, pl.ds(rowbase, RPW)], auxv)

    def e_copy(j, slot):
        return pltpu.make_async_copy(
            embs.at[bi, pl.ds(n0 + j * G, G), 0],
            ebufs[slot], esem.at[slot])

    def o_copy(j, slot):
        return pltpu.make_async_copy(
            obufs[slot], out_hbm.at[pl.ds(rowbase + j * G, G)],
            osem.at[slot])

    def compute(j, slot):
        eb = ebufs[slot]
        ob = obufs[slot]
        cpvs = [auxv[p, pl.ds(j * G, 16)] for p in range(P - 1)]
        for r in range(G):
            crs = [cpvs[p][r] for p in range(P - 1)]
            for k in range(D // 16):
                sl = pl.ds(k * 16, 16)
                acc = eb[r, 0, sl]
                for p in range(1, P):
                    acc = acc + crs[p - 1] * eb[r, p, sl]
                ob[r, sl] = acc

    e_copy(0, 0).start()

    def pair(j2, carry):
        j0 = j2 * 2
        j1 = j0 + 1
        e_copy(j1, 1).start()
        e_copy(j0, 0).wait()

        @pl.when(j2 > 0)
        def _():
            o_copy(j0 - 2, 0).wait()

        compute(j0, 0)
        o_copy(j0, 0).start()
        @pl.when(j1 + 1 < NCH)
        def _():
            e_copy(j1 + 1, 0).start()

        e_copy(j1, 1).wait()

        @pl.when(j2 > 0)
        def _():
            o_copy(j1 - 2, 1).wait()

        compute(j1, 1)
        o_copy(j1, 1).start()
        return carry

    lax.fori_loop(0, NPAIR, pair, 0)
    o_copy(NCH - 2, 0).wait()
    o_copy(NCH - 1, 1).wait()


def _sc_combined(embs, aux):
    mesh = plsc.VectorSubcoreMesh(
        core_axis_name="c", subcore_axis_name="s",
        num_cores=NC, num_subcores=NS)
    return pl.kernel(
        _sc_body,
        out_type=jax.ShapeDtypeStruct((BN, D), jnp.float32),
        mesh=mesh,
        scratch_types=[
            pltpu.VMEM((G, P, D), jnp.float32),
            pltpu.VMEM((G, P, D), jnp.float32),
            pltpu.VMEM((G, D), jnp.float32),
            pltpu.VMEM((G, D), jnp.float32),
            pltpu.VMEM((8, RPW), jnp.float32),
            pltpu.SemaphoreType.DMA((2,)),
            pltpu.SemaphoreType.DMA((2,)),
        ],
        compiler_params=pltpu.CompilerParams(use_tc_tiling_on_sc=True),
    )(embs, aux)


def _tc_body(x_ref, aux_ref, w_ref, b_ref, out_ref):
    aux = aux_ref[...].T
    y = lax.dot_general(x_ref[...], w_ref[...], (((1,), (1,)), ((), ())),
                        preferred_element_type=jnp.float32)
    y = y + aux[:, 5][:, None] * b_ref[...]
    out_ref[0] = jnp.where(aux[:, 6][:, None] > 0, y, 0.0)


@jax.jit
def kernel(embs, vis, W, b, masks):
    visr = vis.reshape(BN, P)
    c = visr[:, 1:].T
    s = 1.0 + jnp.sum(visr[:, 1:], axis=1)[None, :]
    m = masks.reshape(1, BN).astype(jnp.float32)
    aux = jnp.concatenate([c, s, m, jnp.ones((1, BN), jnp.float32)], axis=0)
    comb = _sc_combined(embs, aux)
    b2 = b.reshape(1, O)
    out = pl.pallas_call(
        _tc_body,
        grid=(B, NPB),
        in_specs=[
            pl.BlockSpec((BLK, D), lambda i, j: (i * NPB + j, 0)),
            pl.BlockSpec((8, BLK), lambda i, j: (0, i * NPB + j)),
            pl.BlockSpec((O, D), lambda i, j: (0, 0)),
            pl.BlockSpec((1, O), lambda i, j: (0, 0)),
        ],
        out_specs=pl.BlockSpec((1, BLK, O), lambda i, j: (i, j, 0)),
        out_shape=jax.ShapeDtypeStruct((B, N, O), jnp.float32),
    )(comb, aux, W, b2)
    return out


# speedup vs baseline: 1.3047x; 1.3047x over previous
import jax
import jax.numpy as jnp
from jax import lax
from jax.experimental import pallas as pl
from jax.experimental.pallas import tpu as pltpu
from jax.experimental.pallas import tpu_sc as plsc

B, N, T, P, D, O = 16, 2048, 1, 6, 128, 128
BN = B * N
BLK = 512

NSC = N // 2
SC_ROWS = B * NSC

NC, NS = 2, 16
NW = NC * NS
RPW = SC_ROWS // NW
G = 16
NCH = RPW // G
NPAIR = NCH // 2


def _sc_body(embs, aux_hbm, out_hbm, ebuf0, ebuf1, obuf0, obuf1,
             auxv, esem, osem):
    ci = lax.axis_index("c")
    si = lax.axis_index("s")
    wid = si * NC + ci
    rowbase = wid * RPW
    bi = wid // 2
    n0 = (wid % 2) * RPW

    ebufs = (ebuf0, ebuf1)
    obufs = (obuf0, obuf1)

    pltpu.sync_copy(aux_hbm.at[:, pl.ds(bi * N + n0, RPW)], auxv)

    def e_copy(j, slot):
        return pltpu.make_async_copy(
            embs.at[bi, pl.ds(n0 + j * G, G), 0],
            ebufs[slot], esem.at[slot])

    def o_copy(j, slot):
        return pltpu.make_async_copy(
            obufs[slot], out_hbm.at[pl.ds(rowbase + j * G, G)],
            osem.at[slot])

    def compute(j, slot):
        eb = ebufs[slot]
        ob = obufs[slot]
        cpvs = [auxv[p, pl.ds(j * G, 16)] for p in range(P - 1)]
        for r in range(G):
            crs = [cpvs[p][r] for p in range(P - 1)]
            for k in range(D // 16):
                sl = pl.ds(k * 16, 16)
                acc = eb[r, 0, sl]
                for p in range(1, P):
                    acc = acc + crs[p - 1] * eb[r, p, sl]
                ob[r, sl] = acc

    e_copy(0, 0).start()

    def pair(j2, carry):
        j0 = j2 * 2
        j1 = j0 + 1
        e_copy(j1, 1).start()
        e_copy(j0, 0).wait()

        @pl.when(j2 > 0)
        def _():
            o_copy(j0 - 2, 0).wait()

        compute(j0, 0)
        o_copy(j0, 0).start()

        @pl.when(j1 + 1 < NCH)
        def _():
            e_copy(j1 + 1, 0).start()

        e_copy(j1, 1).wait()

        @pl.when(j2 > 0)
        def _():
            o_copy(j1 - 2, 1).wait()

        compute(j1, 1)
        o_copy(j1, 1).start()
        return carry

    lax.fori_loop(0, NPAIR, pair, 0)
    o_copy(NCH - 2, 0).wait()
    o_copy(NCH - 1, 1).wait()


def _sc_combined(embs, aux):
    mesh = plsc.VectorSubcoreMesh(
        core_axis_name="c", subcore_axis_name="s",
        num_cores=NC, num_subcores=NS)
    return pl.kernel(
        _sc_body,
        out_type=jax.ShapeDtypeStruct((SC_ROWS, D), jnp.float32),
        mesh=mesh,
        scratch_types=[
            pltpu.VMEM((G, P, D), jnp.float32),
            pltpu.VMEM((G, P, D), jnp.float32),
            pltpu.VMEM((G, D), jnp.float32),
            pltpu.VMEM((G, D), jnp.float32),
            pltpu.VMEM((8, RPW), jnp.float32),
            pltpu.SemaphoreType.DMA((2,)),
            pltpu.SemaphoreType.DMA((2,)),
        ],
        compiler_params=pltpu.CompilerParams(use_tc_tiling_on_sc=True),
    )(embs, aux)


NTC_STEPS = B * (N - NSC) // BLK


def _tc_fused_body(embs_hbm, aux_ref, w_ref, b_ref, out_ref, ebuf, sems):
    i = pl.program_id(0)

    def start(step, slot):
        sb = step // 2
        sj = step % 2
        for p in range(P):
            pltpu.make_async_copy(
                embs_hbm.at[sb, pl.ds(NSC + sj * BLK, BLK), 0, p],
                ebuf.at[slot, p],
                sems.at[slot, p],
            ).start()

    def wait(step, slot):
        sb = step // 2
        sj = step % 2
        for p in range(P):
            pltpu.make_async_copy(
                embs_hbm.at[sb, pl.ds(NSC + sj * BLK, BLK), 0, p],
                ebuf.at[slot, p],
                sems.at[slot, p],
            ).wait()

    @pl.when(i == 0)
    def _():
        start(0, 0)

    @pl.when(i + 1 < NTC_STEPS)
    def _():
        start(i + 1, (i + 1) % 2)

    slot = i % 2
    wait(i, slot)

    aux = aux_ref[...].T
    acc = ebuf[slot, 0]
    for p in range(1, P):
        acc += aux[:, p - 1][:, None] * ebuf[slot, p]
    y = lax.dot_general(acc, w_ref[...], (((1,), (1,)), ((), ())),
                        preferred_element_type=jnp.float32)
    y = y + aux[:, 5][:, None] * b_ref[...]
    out_ref[0] = jnp.where(aux[:, 6][:, None] > 0, y, 0.0)


def _tc_fused(embs, aux, W, b2):
    return pl.pallas_call(
        _tc_fused_body,
        grid=(NTC_STEPS,),
        in_specs=[
            pl.BlockSpec(memory_space=pl.ANY),
            pl.BlockSpec((8, BLK), lambda i: (0, (i // 2) * 4 + 2 + i % 2)),
            pl.BlockSpec((O, D), lambda i: (0, 0)),
            pl.BlockSpec((1, O), lambda i: (0, 0)),
        ],
        out_specs=pl.BlockSpec((1, BLK, O), lambda i: (i // 2, i % 2, 0)),
        out_shape=jax.ShapeDtypeStruct((B, N - NSC, O), jnp.float32),
        scratch_shapes=[
            pltpu.VMEM((2, P, BLK, D), jnp.float32),
            pltpu.SemaphoreType.DMA((2, P)),
        ],
    )(embs, aux, W, b2)


NPB_SC = NSC // BLK
NPB_ALL = N // BLK


def _tc_merge_body(x_ref, xtc_ref, aux_ref, w_ref, b_ref, out_ref):
    j = pl.program_id(1)

    @pl.when(j < NPB_SC)
    def _():
        aux = aux_ref[...].T
        y = lax.dot_general(x_ref[...], w_ref[...], (((1,), (1,)), ((), ())),
                            preferred_element_type=jnp.float32)
        y = y + aux[:, 5][:, None] * b_ref[...]
        out_ref[0] = jnp.where(aux[:, 6][:, None] > 0, y, 0.0)

    @pl.when(j >= NPB_SC)
    def _():
        out_ref[0] = xtc_ref[0]


@jax.jit
def kernel(embs, vis, W, b, masks):
    visr = vis.reshape(BN, P)
    c = visr[:, 1:].T
    s = 1.0 + jnp.sum(visr[:, 1:], axis=1)[None, :]
    m = masks.reshape(1, BN).astype(jnp.float32)
    aux = jnp.concatenate([c, s, m, jnp.ones((1, BN), jnp.float32)], axis=0)
    b2 = b.reshape(1, O)

    comb = _sc_combined(embs, aux)
    out_hi = _tc_fused(embs, aux, W, b2)

    out = pl.pallas_call(
        _tc_merge_body,
        grid=(B, NPB_ALL),
        in_specs=[
            pl.BlockSpec((BLK, D),
                         lambda i, j: (i * NPB_SC + jnp.minimum(j, NPB_SC - 1), 0)),
            pl.BlockSpec((1, BLK, O),
                         lambda i, j: (i, jnp.maximum(j - NPB_SC, 0), 0)),
            pl.BlockSpec((8, BLK),
                         lambda i, j: (0, i * NPB_ALL + jnp.minimum(j, NPB_SC - 1))),
            pl.BlockSpec((O, D), lambda i, j: (0, 0)),
            pl.BlockSpec((1, O), lambda i, j: (0, 0)),
        ],
        out_specs=pl.BlockSpec((1, BLK, O), lambda i, j: (i, j, 0)),
        out_shape=jax.ShapeDtypeStruct((B, N, O), jnp.float32),
    )(comb, out_hi, aux, W, b2)
    return out
